# Initial kernel scaffold; baseline (speedup 1.0000x reference)
import jax
import jax.numpy as jnp
from jax.experimental import pallas as pl


def kernel(x):
    raise NotImplementedError("write your pallas kernel here")


# speedup vs baseline: 8.2347x; 8.2347x over previous
import functools
import math

import jax
import jax.numpy as jnp
from jax import lax
from jax.experimental import pallas as pl
from jax.experimental.pallas import tpu as pltpu
from jax.experimental.pallas import tpu_sc as plsc

L = 16
NB = 256
SPARSITY = 0.05


def _kwta_sc(batch, emb, k, n_workers):
  rows_per_w = batch // n_workers
  vregs = emb // L
  UNR = 4
  assert vregs % UNR == 0

  mesh = plsc.VectorSubcoreMesh(core_axis_name="c", subcore_axis_name="s")

  def body(x_hbm, out_hbm, xbuf, hist):
    cid = lax.axis_index("c")
    sid = lax.axis_index("s")
    wid = sid * 2 + cid
    lane = lax.iota(jnp.int32, L)
    ones_i32 = jnp.ones((L,), jnp.int32)

    def ukey_of(raw_f32):
      raw = lax.bitcast_convert_type(raw_f32, jnp.uint32)
      neg = raw >= jnp.uint32(0x80000000)
      flip = jnp.where(neg, jnp.uint32(0xFFFFFFFF), jnp.uint32(0x80000000))
      return raw ^ flip

    def do_row(r, carry_none):
      row = wid * rows_per_w + r
      pltpu.sync_copy(x_hbm.at[row], xbuf)

      prefix = jnp.uint32(0)
      k_rem = jnp.int32(k)

      for level in range(4):
        shift = 24 - 8 * level

        def clr(i, _):
          for u in range(UNR):
            hist[pl.ds((i * UNR + u) * L, L)] = jnp.zeros((L,), jnp.int32)
          return 0
        lax.fori_loop(0, (NB * L) // (L * UNR), clr, 0)

        if level == 0:
          def hpass(i, _):
            for u in range(UNR):
              v = xbuf[pl.ds((i * UNR + u) * L, L)]
              uk = ukey_of(v)
              bucket = lax.convert_element_type(
                  lax.shift_right_logical(uk, jnp.uint32(shift)), jnp.int32)
              idx = bucket * L + lane
              plsc.addupdate_scatter(hist, [idx], ones_i32)
            return 0
        else:
          hi_shift = jnp.uint32(shift + 8)
          pref_hi = lax.shift_right_logical(prefix, hi_shift)

          def hpass(i, _):
            for u in range(UNR):
              v = xbuf[pl.ds((i * UNR + u) * L, L)]
              uk = ukey_of(v)
              match = lax.shift_right_logical(uk, hi_shift) == pref_hi
              bucket = lax.convert_element_type(
                  jnp.uint32(0xFF)
                  & lax.shift_right_logical(uk, jnp.uint32(shift)), jnp.int32)
              idx = bucket * L + lane
              plsc.addupdate_scatter(hist, [idx], ones_i32, mask=match)
            return 0
        lax.fori_loop(0, vregs // UNR, hpass, 0)

        carry = jnp.int32(0)
        bucket_sel = jnp.int32(0)
        k_next = jnp.int32(0)
        for g in range(NB // L - 1, -1, -1):
          tg = jnp.zeros((L,), jnp.int32)
          for sub in range(L):
            gidx = g * NB + lane * L + sub
            tg = tg + plsc.load_gather(hist, [gidx])
          rc = plsc.cumsum(lax.rev(tg, (0,)))
          srev = lax.rev(rc, (0,))
          group_sum = jnp.max(rc)
          inc = srev + carry
          ge = inc >= k_rem
          cnt = jnp.max(plsc.all_reduce_population_count(ge))
          jstar = cnt - 1
          t_at = jnp.sum(jnp.where(lane == jstar, tg, 0))
          inc_at = jnp.sum(jnp.where(lane == jstar, inc, 0))
          cond = (carry < k_rem) & (carry + group_sum >= k_rem)
          bucket_sel = bucket_sel + jnp.where(cond, g * L + jstar, 0)
          k_next = k_next + jnp.where(cond, k_rem - (inc_at - t_at), 0)
          carry = carry + group_sum
        k_rem = k_next
        prefix = prefix | lax.shift_left(
            lax.convert_element_type(bucket_sel, jnp.uint32),
            jnp.uint32(shift))

      def mpass(i, _):
        for u in range(UNR):
          sl = pl.ds((i * UNR + u) * L, L)
          uk = ukey_of(xbuf[sl])
          xbuf[sl] = jnp.where(uk >= prefix, jnp.float32(1.0),
                               jnp.float32(0.0))
        return 0
      lax.fori_loop(0, vregs // UNR, mpass, 0)

      pltpu.sync_copy(xbuf, out_hbm.at[row])
      return 0

    lax.fori_loop(0, rows_per_w, do_row, 0)

  return pl.kernel(
      body,
      out_type=jax.ShapeDtypeStruct((batch, emb), jnp.float32),
      mesh=mesh,
      compiler_params=pltpu.CompilerParams(needs_layout_passes=False),
      scratch_types=[
          pltpu.VMEM((emb,), jnp.float32),
          pltpu.VMEM((NB * L,), jnp.int32),
      ],
  )


@jax.jit
def kernel(x):
  batch, emb = x.shape
  k = math.ceil(SPARSITY * emb)
  return _kwta_sc(batch, emb, k, 32)(x)


# speedup vs baseline: 26.4830x; 3.2160x over previous
import functools
import math

import jax
import jax.numpy as jnp
from jax import lax
from jax.experimental import pallas as pl
from jax.experimental.pallas import tpu as pltpu
from jax.experimental.pallas import tpu_sc as plsc

L = 16
NB = 256
SPARSITY = 0.05


def _kwta_sc(batch, emb, k, n_workers):
  rows_per_w = batch // n_workers
  vregs = emb // L
  UNR = 4
  assert vregs % UNR == 0

  mesh = plsc.VectorSubcoreMesh(core_axis_name="c", subcore_axis_name="s")

  def body(x_hbm, out_hbm, xbuf, hist):
    cid = lax.axis_index("c")
    sid = lax.axis_index("s")
    wid = sid * 2 + cid
    lane = lax.iota(jnp.int32, L)
    ones_i32 = jnp.ones((L,), jnp.int32)

    def ukey_of(raw_f32):
      raw = lax.bitcast_convert_type(raw_f32, jnp.uint32)
      neg = raw >= jnp.uint32(0x80000000)
      flip = jnp.where(neg, jnp.uint32(0xFFFFFFFF), jnp.uint32(0x80000000))
      return raw ^ flip

    def do_row(r, carry_none):
      row = wid * rows_per_w + r
      pltpu.sync_copy(x_hbm.at[row], xbuf)

      prefix = jnp.uint32(0)
      k_rem = jnp.int32(k)

      for level in range(4):
        shift = 24 - 8 * level

        @plsc.parallel_loop(0, NB * L, L, unroll=UNR)
        def _(i):
          hist[pl.ds(i, L)] = jnp.zeros((L,), jnp.int32)

        if level == 0:
          @plsc.parallel_loop(0, emb, L, unroll=UNR)
          def _(i):
            uk = ukey_of(xbuf[pl.ds(i, L)])
            bucket = lax.convert_element_type(
                lax.shift_right_logical(uk, jnp.uint32(shift)), jnp.int32)
            idx = bucket * L + lane
            plsc.addupdate_scatter(hist, [idx], ones_i32)
        else:
          hi_shift = jnp.uint32(shift + 8)
          pref_hi = lax.shift_right_logical(prefix, hi_shift)

          @plsc.parallel_loop(0, emb, L, unroll=UNR)
          def _(i):
            uk = ukey_of(xbuf[pl.ds(i, L)])
            match = lax.shift_right_logical(uk, hi_shift) == pref_hi
            bucket = lax.convert_element_type(
                jnp.uint32(0xFF)
                & lax.shift_right_logical(uk, jnp.uint32(shift)), jnp.int32)
            idx = bucket * L + lane
            plsc.addupdate_scatter(hist, [idx], ones_i32, mask=match)

        carry = jnp.int32(0)
        bucket_sel = jnp.int32(0)
        k_next = jnp.int32(0)
        for g in range(NB // L - 1, -1, -1):
          tg = jnp.zeros((L,), jnp.int32)
          for sub in range(L):
            gidx = g * NB + lane * L + sub
            tg = tg + plsc.load_gather(hist, [gidx])
          rc = plsc.cumsum(lax.rev(tg, (0,)))
          srev = lax.rev(rc, (0,))
          group_sum = jnp.max(rc)
          inc = srev + carry
          ge = inc >= k_rem
          cnt = jnp.max(plsc.all_reduce_population_count(ge))
          jstar = cnt - 1
          t_at = jnp.sum(jnp.where(lane == jstar, tg, 0))
          inc_at = jnp.sum(jnp.where(lane == jstar, inc, 0))
          cond = (carry < k_rem) & (carry + group_sum >= k_rem)
          bucket_sel = bucket_sel + jnp.where(cond, g * L + jstar, 0)
          k_next = k_next + jnp.where(cond, k_rem - (inc_at - t_at), 0)
          carry = carry + group_sum
        k_rem = k_next
        prefix = prefix | lax.shift_left(
            lax.convert_element_type(bucket_sel, jnp.uint32),
            jnp.uint32(shift))

      @plsc.parallel_loop(0, emb, L, unroll=UNR)
      def _(i):
        sl = pl.ds(i, L)
        uk = ukey_of(xbuf[sl])
        xbuf[sl] = jnp.where(uk >= prefix, jnp.float32(1.0),
                             jnp.float32(0.0))

      pltpu.sync_copy(xbuf, out_hbm.at[row])
      return 0

    lax.fori_loop(0, rows_per_w, do_row, 0)

  return pl.kernel(
      body,
      out_type=jax.ShapeDtypeStruct((batch, emb), jnp.float32),
      mesh=mesh,
      compiler_params=pltpu.CompilerParams(needs_layout_passes=False),
      scratch_types=[
          pltpu.VMEM((emb,), jnp.float32),
          pltpu.VMEM((NB * L,), jnp.int32),
      ],
  )


@jax.jit
def kernel(x):
  batch, emb = x.shape
  k = math.ceil(SPARSITY * emb)
  return _kwta_sc(batch, emb, k, 32)(x)


# speedup vs baseline: 27.2875x; 1.0304x over previous
import functools
import math

import jax
import jax.numpy as jnp
from jax import lax
from jax.experimental import pallas as pl
from jax.experimental.pallas import tpu as pltpu
from jax.experimental.pallas import tpu_sc as plsc

L = 16
NB = 256
SPARSITY = 0.05


def _kwta_sc(batch, emb, k, n_workers):
  rows_per_w = batch // n_workers
  vregs = emb // L
  UNR = 8
  assert vregs % UNR == 0

  mesh = plsc.VectorSubcoreMesh(core_axis_name="c", subcore_axis_name="s")

  def body(x_hbm, out_hbm, xbuf, hist):
    cid = lax.axis_index("c")
    sid = lax.axis_index("s")
    wid = sid * 2 + cid
    lane = lax.iota(jnp.int32, L)
    ones_i32 = jnp.ones((L,), jnp.int32)

    def ukey_of(raw_f32):
      raw = lax.bitcast_convert_type(raw_f32, jnp.uint32)
      neg = raw >= jnp.uint32(0x80000000)
      flip = jnp.where(neg, jnp.uint32(0xFFFFFFFF), jnp.uint32(0x80000000))
      return raw ^ flip

    def do_row(r, carry_none):
      row = wid * rows_per_w + r
      pltpu.sync_copy(x_hbm.at[row], xbuf)

      prefix = jnp.uint32(0)
      k_rem = jnp.int32(k)

      for level in range(4):
        shift = 24 - 8 * level

        @plsc.parallel_loop(0, NB * L, L, unroll=UNR)
        def _(i):
          hist[pl.ds(i, L)] = jnp.zeros((L,), jnp.int32)

        if level == 0:
          @plsc.parallel_loop(0, emb, L, unroll=UNR)
          def _(i):
            uk = ukey_of(xbuf[pl.ds(i, L)])
            bucket = lax.convert_element_type(
                lax.shift_right_logical(uk, jnp.uint32(shift)), jnp.int32)
            idx = bucket * L + lane
            plsc.addupdate_scatter(hist, [idx], ones_i32)
        else:
          hi_shift = jnp.uint32(shift + 8)
          pref_hi = lax.shift_right_logical(prefix, hi_shift)

          @plsc.parallel_loop(0, emb, L, unroll=UNR)
          def _(i):
            uk = ukey_of(xbuf[pl.ds(i, L)])
            match = lax.shift_right_logical(uk, hi_shift) == pref_hi
            bucket = lax.convert_element_type(
                jnp.uint32(0xFF)
                & lax.shift_right_logical(uk, jnp.uint32(shift)), jnp.int32)
            idx = bucket * L + lane
            plsc.addupdate_scatter(hist, [idx], ones_i32, mask=match)

        carry = jnp.int32(0)
        bucket_sel = jnp.int32(0)
        k_next = jnp.int32(0)
        for g in range(NB // L - 1, -1, -1):
          tg = jnp.zeros((L,), jnp.int32)
          for sub in range(L):
            gidx = g * NB + lane * L + sub
            tg = tg + plsc.load_gather(hist, [gidx])
          rc = plsc.cumsum(lax.rev(tg, (0,)))
          srev = lax.rev(rc, (0,))
          group_sum = jnp.max(rc)
          inc = srev + carry
          ge = inc >= k_rem
          cnt = jnp.max(plsc.all_reduce_population_count(ge))
          jstar = cnt - 1
          t_at = jnp.sum(jnp.where(lane == jstar, tg, 0))
          inc_at = jnp.sum(jnp.where(lane == jstar, inc, 0))
          cond = (carry < k_rem) & (carry + group_sum >= k_rem)
          bucket_sel = bucket_sel + jnp.where(cond, g * L + jstar, 0)
          k_next = k_next + jnp.where(cond, k_rem - (inc_at - t_at), 0)
          carry = carry + group_sum
        k_rem = k_next
        prefix = prefix | lax.shift_left(
            lax.convert_element_type(bucket_sel, jnp.uint32),
            jnp.uint32(shift))

      @plsc.parallel_loop(0, emb, L, unroll=UNR)
      def _(i):
        sl = pl.ds(i, L)
        uk = ukey_of(xbuf[sl])
        xbuf[sl] = jnp.where(uk >= prefix, jnp.float32(1.0),
                             jnp.float32(0.0))

      pltpu.sync_copy(xbuf, out_hbm.at[row])
      return 0

    lax.fori_loop(0, rows_per_w, do_row, 0)

  return pl.kernel(
      body,
      out_type=jax.ShapeDtypeStruct((batch, emb), jnp.float32),
      mesh=mesh,
      compiler_params=pltpu.CompilerParams(needs_layout_passes=False),
      scratch_types=[
          pltpu.VMEM((emb,), jnp.float32),
          pltpu.VMEM((NB * L,), jnp.int32),
      ],
  )


@jax.jit
def kernel(x):
  batch, emb = x.shape
  k = math.ceil(SPARSITY * emb)
  return _kwta_sc(batch, emb, k, 32)(x)
